# Initial kernel scaffold; baseline (speedup 1.0000x reference)
import jax
import jax.numpy as jnp
from jax.experimental import pallas as pl


def kernel(input_ids, embed, layers_0_q_proj_weight, layers_0_q_proj_lora_A, layers_0_q_proj_lora_B, layers_0_v_proj_weight, layers_0_v_proj_lora_A, layers_0_v_proj_lora_B, layers_1_q_proj_weight, layers_1_q_proj_lora_A, layers_1_q_proj_lora_B, layers_1_v_proj_weight, layers_1_v_proj_lora_A, layers_1_v_proj_lora_B, layers_2_q_proj_weight, layers_2_q_proj_lora_A, layers_2_q_proj_lora_B, layers_2_v_proj_weight, layers_2_v_proj_lora_A, layers_2_v_proj_lora_B, layers_3_q_proj_weight, layers_3_q_proj_lora_A, layers_3_q_proj_lora_B, layers_3_v_proj_weight, layers_3_v_proj_lora_A, layers_3_v_proj_lora_B):
    raise NotImplementedError("write your pallas kernel here")


# speedup vs baseline: 1.5324x; 1.5324x over previous
import functools

import jax
import jax.numpy as jnp
from jax.experimental import pallas as pl
from jax.experimental.pallas import tpu as pltpu


def _fused4_kernel(n_lora, *refs):
    x_ref = refs[0]
    ws = refs[1:5]
    las = refs[5:5 + n_lora]
    lbs = refs[5 + n_lora:5 + 2 * n_lora]
    y_ref = refs[-1]
    h = x_ref[...]
    for m in range(4):
        acc = jnp.dot(h, ws[m][...], preferred_element_type=jnp.float32)
        if m < n_lora:
            xa = jnp.dot(h, las[m][...], preferred_element_type=jnp.float32)
            acc += jnp.dot(xa, lbs[m][...].astype(jnp.float32),
                           preferred_element_type=jnp.float32)
        h = (h.astype(jnp.float32) + acc).astype(h.dtype)
    y_ref[...] = h


def _fused4(x, ws, lora_as, lora_bs, *, tm=512):
    M, H = x.shape
    n_lora = len(lora_as)
    tm = min(tm, M)
    grid = (M // tm,)

    full = lambda shape: pl.BlockSpec(shape, lambda i: (0,) * len(shape))
    in_specs = [pl.BlockSpec((tm, H), lambda i: (i, 0))]
    in_specs += [full((H, H))] * 4
    in_specs += [full(a.shape) for a in lora_as]
    in_specs += [full(b.shape) for b in lora_bs]

    R = lora_as[0].shape[1] if lora_as else 0
    cost = pl.CostEstimate(
        flops=4 * 2 * M * H * H + n_lora * (2 * M * H * R + 2 * M * R * H),
        transcendentals=0,
        bytes_accessed=2 * (2 * M * H + 4 * H * H))

    return pl.pallas_call(
        functools.partial(_fused4_kernel, n_lora),
        out_shape=jax.ShapeDtypeStruct((M, H), x.dtype),
        grid=grid,
        in_specs=in_specs,
        out_specs=pl.BlockSpec((tm, H), lambda i: (i, 0)),
        compiler_params=pltpu.CompilerParams(
            dimension_semantics=("parallel",),
            vmem_limit_bytes=100 * 1024 * 1024),
        cost_estimate=cost,
    )(x, *ws, *lora_as, *lora_bs)


def kernel(input_ids, embed, layers_0_q_proj_weight, layers_0_q_proj_lora_A, layers_0_q_proj_lora_B, layers_0_v_proj_weight, layers_0_v_proj_lora_A, layers_0_v_proj_lora_B, layers_1_q_proj_weight, layers_1_q_proj_lora_A, layers_1_q_proj_lora_B, layers_1_v_proj_weight, layers_1_v_proj_lora_A, layers_1_v_proj_lora_B, layers_2_q_proj_weight, layers_2_q_proj_lora_A, layers_2_q_proj_lora_B, layers_2_v_proj_weight, layers_2_v_proj_lora_A, layers_2_v_proj_lora_B, layers_3_q_proj_weight, layers_3_q_proj_lora_A, layers_3_q_proj_lora_B, layers_3_v_proj_weight, layers_3_v_proj_lora_A, layers_3_v_proj_lora_B):
    B, S = input_ids.shape
    H = embed.shape[1]
    h = embed[input_ids].reshape(B * S, H)

    h = _fused4(
        h,
        [layers_0_q_proj_weight.T, layers_0_v_proj_weight.T,
         layers_1_q_proj_weight.T, layers_1_v_proj_weight.T],
        [layers_0_q_proj_lora_A, layers_0_v_proj_lora_A,
         layers_1_q_proj_lora_A, layers_1_v_proj_lora_A],
        [layers_0_q_proj_lora_B, layers_0_v_proj_lora_B,
         layers_1_q_proj_lora_B, layers_1_v_proj_lora_B])

    h = _fused4(
        h,
        [layers_2_q_proj_weight.T, layers_2_v_proj_weight.T,
         layers_3_q_proj_weight.T, layers_3_v_proj_weight.T],
        [layers_2_q_proj_lora_A, layers_2_v_proj_lora_A],
        [layers_2_q_proj_lora_B, layers_2_v_proj_lora_B])

    return h.reshape(B, S, H)


# speedup vs baseline: 1.5639x; 1.0206x over previous
import functools

import jax
import jax.numpy as jnp
from jax.experimental import pallas as pl
from jax.experimental.pallas import tpu as pltpu


def _fused4_kernel(n_lora, *refs):
    x_ref = refs[0]
    ws = refs[1:5]
    las = refs[5:5 + n_lora]
    lbs = refs[5 + n_lora:5 + 2 * n_lora]
    y_ref = refs[-1]
    h = x_ref[...]
    for m in range(4):
        acc = jax.lax.dot_general(
            h, ws[m][...], (((1,), (1,)), ((), ())),
            preferred_element_type=jnp.float32)
        if m < n_lora:
            xa = jnp.dot(h, las[m][...], preferred_element_type=jnp.float32)
            acc += jnp.dot(xa, lbs[m][...].astype(jnp.float32),
                           preferred_element_type=jnp.float32)
        h = (h.astype(jnp.float32) + acc).astype(h.dtype)
    y_ref[...] = h


def _fused4(x, ws, lora_as, lora_bs, *, tm=512):
    M, H = x.shape
    n_lora = len(lora_as)
    tm = min(tm, M)
    grid = (M // tm,)

    full = lambda shape: pl.BlockSpec(shape, lambda i: (0,) * len(shape))
    in_specs = [pl.BlockSpec((tm, H), lambda i: (i, 0))]
    in_specs += [full((H, H))] * 4
    in_specs += [full(a.shape) for a in lora_as]
    in_specs += [full(b.shape) for b in lora_bs]

    R = lora_as[0].shape[1] if lora_as else 0
    cost = pl.CostEstimate(
        flops=4 * 2 * M * H * H + n_lora * (2 * M * H * R + 2 * M * R * H),
        transcendentals=0,
        bytes_accessed=2 * (2 * M * H + 4 * H * H))

    return pl.pallas_call(
        functools.partial(_fused4_kernel, n_lora),
        out_shape=jax.ShapeDtypeStruct((M, H), x.dtype),
        grid=grid,
        in_specs=in_specs,
        out_specs=pl.BlockSpec((tm, H), lambda i: (i, 0)),
        compiler_params=pltpu.CompilerParams(
            dimension_semantics=("parallel",),
            vmem_limit_bytes=100 * 1024 * 1024),
        cost_estimate=cost,
    )(x, *ws, *lora_as, *lora_bs)


def kernel(input_ids, embed, layers_0_q_proj_weight, layers_0_q_proj_lora_A, layers_0_q_proj_lora_B, layers_0_v_proj_weight, layers_0_v_proj_lora_A, layers_0_v_proj_lora_B, layers_1_q_proj_weight, layers_1_q_proj_lora_A, layers_1_q_proj_lora_B, layers_1_v_proj_weight, layers_1_v_proj_lora_A, layers_1_v_proj_lora_B, layers_2_q_proj_weight, layers_2_q_proj_lora_A, layers_2_q_proj_lora_B, layers_2_v_proj_weight, layers_2_v_proj_lora_A, layers_2_v_proj_lora_B, layers_3_q_proj_weight, layers_3_q_proj_lora_A, layers_3_q_proj_lora_B, layers_3_v_proj_weight, layers_3_v_proj_lora_A, layers_3_v_proj_lora_B):
    B, S = input_ids.shape
    H = embed.shape[1]
    h = embed[input_ids].reshape(B * S, H)

    h = _fused4(
        h,
        [layers_0_q_proj_weight, layers_0_v_proj_weight,
         layers_1_q_proj_weight, layers_1_v_proj_weight],
        [layers_0_q_proj_lora_A, layers_0_v_proj_lora_A,
         layers_1_q_proj_lora_A, layers_1_v_proj_lora_A],
        [layers_0_q_proj_lora_B, layers_0_v_proj_lora_B,
         layers_1_q_proj_lora_B, layers_1_v_proj_lora_B])

    h = _fused4(
        h,
        [layers_2_q_proj_weight, layers_2_v_proj_weight,
         layers_3_q_proj_weight, layers_3_v_proj_weight],
        [layers_2_q_proj_lora_A, layers_2_v_proj_lora_A],
        [layers_2_q_proj_lora_B, layers_2_v_proj_lora_B])

    return h.reshape(B, S, H)


# speedup vs baseline: 1.5803x; 1.0105x over previous
import functools

import jax
import jax.numpy as jnp
from jax.experimental import pallas as pl
from jax.experimental.pallas import tpu as pltpu


def _fused4_kernel(n_lora, *refs):
    x_ref = refs[0]
    ws = refs[1:5]
    las = refs[5:5 + n_lora]
    lbs = refs[5 + n_lora:5 + 2 * n_lora]
    y_ref = refs[-1]
    half = x_ref.shape[0] // 2
    hs = [x_ref[:half, :], x_ref[half:, :]]
    for m in range(4):
        w = ws[m][...]
        b = lbs[m][...].astype(jnp.float32) if m < n_lora else None
        for c in range(2):
            h = hs[c]
            acc = jax.lax.dot_general(
                h, w, (((1,), (1,)), ((), ())),
                preferred_element_type=jnp.float32)
            if m < n_lora:
                xa = jnp.dot(h, las[m][...],
                             preferred_element_type=jnp.float32)
                acc += jnp.dot(xa, b, preferred_element_type=jnp.float32)
            hs[c] = (h.astype(jnp.float32) + acc).astype(h.dtype)
    y_ref[:half, :] = hs[0]
    y_ref[half:, :] = hs[1]


def _fused4(x, ws, lora_as, lora_bs, *, tm=512):
    M, H = x.shape
    n_lora = len(lora_as)
    tm = min(tm, M)
    grid = (M // tm,)

    full = lambda shape: pl.BlockSpec(shape, lambda i: (0,) * len(shape))
    in_specs = [pl.BlockSpec((tm, H), lambda i: (i, 0))]
    in_specs += [full((H, H))] * 4
    in_specs += [full(a.shape) for a in lora_as]
    in_specs += [full(b.shape) for b in lora_bs]

    R = lora_as[0].shape[1] if lora_as else 0
    cost = pl.CostEstimate(
        flops=4 * 2 * M * H * H + n_lora * (2 * M * H * R + 2 * M * R * H),
        transcendentals=0,
        bytes_accessed=2 * (2 * M * H + 4 * H * H))

    return pl.pallas_call(
        functools.partial(_fused4_kernel, n_lora),
        out_shape=jax.ShapeDtypeStruct((M, H), x.dtype),
        grid=grid,
        in_specs=in_specs,
        out_specs=pl.BlockSpec((tm, H), lambda i: (i, 0)),
        compiler_params=pltpu.CompilerParams(
            dimension_semantics=("parallel",),
            vmem_limit_bytes=100 * 1024 * 1024),
        cost_estimate=cost,
    )(x, *ws, *lora_as, *lora_bs)


def kernel(input_ids, embed, layers_0_q_proj_weight, layers_0_q_proj_lora_A, layers_0_q_proj_lora_B, layers_0_v_proj_weight, layers_0_v_proj_lora_A, layers_0_v_proj_lora_B, layers_1_q_proj_weight, layers_1_q_proj_lora_A, layers_1_q_proj_lora_B, layers_1_v_proj_weight, layers_1_v_proj_lora_A, layers_1_v_proj_lora_B, layers_2_q_proj_weight, layers_2_q_proj_lora_A, layers_2_q_proj_lora_B, layers_2_v_proj_weight, layers_2_v_proj_lora_A, layers_2_v_proj_lora_B, layers_3_q_proj_weight, layers_3_q_proj_lora_A, layers_3_q_proj_lora_B, layers_3_v_proj_weight, layers_3_v_proj_lora_A, layers_3_v_proj_lora_B):
    B, S = input_ids.shape
    H = embed.shape[1]
    h = embed[input_ids].reshape(B * S, H)

    h = _fused4(
        h,
        [layers_0_q_proj_weight, layers_0_v_proj_weight,
         layers_1_q_proj_weight, layers_1_v_proj_weight],
        [layers_0_q_proj_lora_A, layers_0_v_proj_lora_A,
         layers_1_q_proj_lora_A, layers_1_v_proj_lora_A],
        [layers_0_q_proj_lora_B, layers_0_v_proj_lora_B,
         layers_1_q_proj_lora_B, layers_1_v_proj_lora_B])

    h = _fused4(
        h,
        [layers_2_q_proj_weight, layers_2_v_proj_weight,
         layers_3_q_proj_weight, layers_3_v_proj_weight],
        [layers_2_q_proj_lora_A, layers_2_v_proj_lora_A],
        [layers_2_q_proj_lora_B, layers_2_v_proj_lora_B])

    return h.reshape(B, S, H)
